# Initial kernel scaffold; baseline (speedup 1.0000x reference)
import jax
import jax.numpy as jnp
from jax import lax
from jax.experimental import pallas as pl
from jax.experimental.pallas import tpu as pltpu
from jax.experimental.pallas import tpu_sc as plsc

N_NODES = 50000
NW = 32
RPW = 1568
NP = NW * RPW
G = 128
NEG = -3.0e38
BLK = 512

_HI = lax.Precision.HIGHEST


def _dot(a, b):
    return jnp.dot(a, b, preferred_element_type=jnp.float32, precision=_HI)


def _uv_body(a_ref, wd_ref, wb_ref, u_ref, v_ref):
    a = a_ref[...]
    u_ref[...] = _dot(a, wd_ref[...])
    v_ref[...] = _dot(a, wb_ref[...])


def _tc_uv(a, wd, wb):
    n, k = a.shape
    return pl.pallas_call(
        _uv_body,
        grid=(n // BLK,),
        in_specs=[pl.BlockSpec((BLK, k), lambda i: (i, 0)),
                  pl.BlockSpec((k, 64), lambda i: (0, 0)),
                  pl.BlockSpec((k, 64), lambda i: (0, 0))],
        out_specs=[pl.BlockSpec((BLK, 64), lambda i: (i, 0)),
                   pl.BlockSpec((BLK, 64), lambda i: (i, 0))],
        out_shape=[jax.ShapeDtypeStruct((n, 64), jnp.float32)] * 2,
    )(a, wd, wb)


def _mid_body(u_ref, m_ref, b_ref, wd_ref, wb_ref, uo_ref, vo_ref):
    a = jnp.maximum(u_ref[...] + b_ref[0:1, :] + m_ref[...], 0.0)
    uo_ref[...] = _dot(a, wd_ref[...])
    vo_ref[...] = _dot(a, wb_ref[...])


def _tc_mid(u, m, b2d, wd, wb):
    return pl.pallas_call(
        _mid_body,
        grid=(NP // BLK,),
        in_specs=[pl.BlockSpec((BLK, 64), lambda i: (i, 0)),
                  pl.BlockSpec((BLK, 64), lambda i: (i, 0)),
                  pl.BlockSpec((8, 64), lambda i: (0, 0)),
                  pl.BlockSpec((64, 64), lambda i: (0, 0)),
                  pl.BlockSpec((64, 64), lambda i: (0, 0))],
        out_specs=[pl.BlockSpec((BLK, 64), lambda i: (i, 0)),
                   pl.BlockSpec((BLK, 64), lambda i: (i, 0))],
        out_shape=[jax.ShapeDtypeStruct((NP, 64), jnp.float32)] * 2,
    )(u, m, b2d, wd, wb)


def _x5g_body(u_ref, m_ref, b_ref, x5_ref, g_ref):
    i = pl.program_id(0)
    x5 = jnp.maximum(u_ref[...] + b_ref[0:1, :] + m_ref[...], 0.0)
    x5_ref[...] = x5
    pm = jnp.broadcast_to(jnp.max(x5, axis=0, keepdims=True), (8, 64))

    @pl.when(i == 0)
    def _():
        g_ref[...] = pm

    @pl.when(i > 0)
    def _():
        g_ref[...] = jnp.maximum(g_ref[...], pm)


def _tc_x5g(u, m, b2d):
    return pl.pallas_call(
        _x5g_body,
        grid=(NP // BLK,),
        in_specs=[pl.BlockSpec((BLK, 64), lambda i: (i, 0)),
                  pl.BlockSpec((BLK, 64), lambda i: (i, 0)),
                  pl.BlockSpec((8, 64), lambda i: (0, 0))],
        out_specs=[pl.BlockSpec((BLK, 64), lambda i: (i, 0)),
                   pl.BlockSpec((8, 64), lambda i: (0, 0))],
        out_shape=[jax.ShapeDtypeStruct((NP, 64), jnp.float32),
                   jax.ShapeDtypeStruct((8, 64), jnp.float32)],
    )(u, m, b2d)


def _fin_body(x5_ref, g_ref, xp_ref, a_ref, bm_ref, bl1_ref, wl2_ref,
              bl2_ref, out_ref):
    gb = _dot(g_ref[0:1, :], bm_ref[...])
    h = jnp.maximum(_dot(x5_ref[...], a_ref[...]) + gb + bl1_ref[0:1, :], 0.0)
    out_ref[...] = xp_ref[...] + _dot(h, wl2_ref[...]) + bl2_ref[0:1, :]


def _tc_final(x5, g8, xpad, amat, bmat, bl1_2d, wl2p, bl2_2d):
    return pl.pallas_call(
        _fin_body,
        grid=(NP // BLK,),
        in_specs=[pl.BlockSpec((BLK, 64), lambda i: (i, 0)),
                  pl.BlockSpec((8, 64), lambda i: (0, 0)),
                  pl.BlockSpec((BLK, 8), lambda i: (i, 0)),
                  pl.BlockSpec((64, 128), lambda i: (0, 0)),
                  pl.BlockSpec((64, 128), lambda i: (0, 0)),
                  pl.BlockSpec((8, 128), lambda i: (0, 0)),
                  pl.BlockSpec((128, 8), lambda i: (0, 0)),
                  pl.BlockSpec((8, 8), lambda i: (0, 0))],
        out_specs=pl.BlockSpec((BLK, 8), lambda i: (i, 0)),
        out_shape=jax.ShapeDtypeStruct((NP, 8), jnp.float32),
    )(x5, g8, xpad, amat, bmat, bl1_2d, wl2p, bl2_2d)


def _segmax_body(v_hbm, src_hbm, dst_hbm, st_hbm, m_hbm,
                 idx_v, rows_v, acc_v, dst_sm, st_sm, sem):
    wid = lax.axis_index("s") * 2 + lax.axis_index("c")
    lo = wid * RPW
    hi = lo + RPW
    pltpu.sync_copy(st_hbm, st_sm)

    neg = jnp.full((16,), NEG, jnp.float32)

    @pl.loop(0, RPW)
    def _(r):
        for f in range(4):
            acc_v[r, pl.ds(f * 16, 16)] = neg

    s = st_sm[wid]
    e = st_sm[wid + 1]
    c0 = s // G
    c1 = lax.div(e + G - 1, G)

    def chunk(ci, carry):
        base = ci * G
        pltpu.sync_copy(src_hbm.at[pl.ds(base, G)], idx_v)
        pltpu.sync_copy(dst_hbm.at[pl.ds(base, G)], dst_sm)
        pltpu.async_copy(v_hbm.at[idx_v], rows_v, sem).wait()

        def edge(ei, carry2):
            d = dst_sm[ei]

            @pl.when((d >= lo) & (d < hi))
            def _():
                dl = d - lo
                for f in range(4):
                    sl = pl.ds(f * 16, 16)
                    acc_v[dl, sl] = jnp.maximum(acc_v[dl, sl], rows_v[ei, sl])

            return carry2

        lax.fori_loop(0, G, edge, 0)
        return carry

    lax.fori_loop(c0, c1, chunk, 0)
    pltpu.sync_copy(acc_v, m_hbm.at[pl.ds(lo, RPW)])


def _sc_segmax(v, src_s, dst_s, starts):
    mesh = plsc.VectorSubcoreMesh(core_axis_name="c", subcore_axis_name="s")
    kfn = pl.kernel(
        _segmax_body,
        out_type=jax.ShapeDtypeStruct((NP, 64), jnp.float32),
        mesh=mesh,
        scratch_types=[
            pltpu.VMEM((G,), jnp.int32),
            pltpu.VMEM((G, 64), jnp.float32),
            pltpu.VMEM((RPW, 64), jnp.float32),
            pltpu.SMEM((G,), jnp.int32),
            pltpu.SMEM((64,), jnp.int32),
            pltpu.SemaphoreType.DMA,
        ],
    )
    return kfn(v, src_s, dst_s, starts)


def kernel(x, edge_index, W1, b1, W2, b2, W3, b3, W4, b4, W5, b5,
           Wl1, bl1, Wl2, bl2):
    f32 = jnp.float32
    src = edge_index[0].astype(jnp.int32)
    dst = edge_index[1].astype(jnp.int32)

    dst_s, src_s = lax.sort((dst, src), num_keys=1)
    bounds = jnp.arange(0, NW + 1, dtype=jnp.int32) * RPW
    starts = jnp.searchsorted(dst_s, bounds).astype(jnp.int32)
    starts = jnp.concatenate([starts, jnp.zeros((31,), jnp.int32)])

    xpad = jnp.zeros((NP, 8), f32).at[:N_NODES, :3].set(x)

    wd1 = jnp.zeros((8, 64), f32).at[:3].set(W1[:3] - W1[3:])
    wb1 = jnp.zeros((8, 64), f32).at[:3].set(W1[3:])
    u, v = _tc_uv(xpad, wd1, wb1)
    m = _sc_segmax(v, src_s, dst_s, starts)
    bprev = b1

    for (W, b) in ((W2, b2), (W3, b3), (W4, b4), (W5, b5)):
        wd = W[:64] - W[64:]
        wb = W[64:]
        b2d = jnp.broadcast_to(bprev.reshape(1, 64), (8, 64))
        u, v = _tc_mid(u, m, b2d, wd, wb)
        m = _sc_segmax(v, src_s, dst_s, starts)
        bprev = b

    b2d5 = jnp.broadcast_to(bprev.reshape(1, 64), (8, 64))
    x5, g8 = _tc_x5g(u, m, b2d5)

    amat = Wl1[:64]
    bmat = Wl1[64:]
    bl1_2d = jnp.broadcast_to(bl1.reshape(1, 128), (8, 128))
    wl2p = jnp.zeros((128, 8), f32).at[:, :3].set(Wl2)
    bl2_2d = jnp.zeros((8, 8), f32).at[:, :3].set(
        jnp.broadcast_to(bl2.reshape(1, 3), (8, 3)))
    outp = _tc_final(x5, g8, xpad, amat, bmat, bl1_2d, wl2p, bl2_2d)
    return outp[:N_NODES, :3]


# speedup vs baseline: 4.5519x; 4.5519x over previous
import jax
import jax.numpy as jnp
from jax import lax
from jax.experimental import pallas as pl
from jax.experimental.pallas import tpu as pltpu
from jax.experimental.pallas import tpu_sc as plsc

N_NODES = 50000
NW = 32
RPW = 1568
NP = NW * RPW
G = 128
NEG = -3.0e38
BLK = 512

_HI = lax.Precision.HIGHEST


def _dot(a, b):
    return jnp.dot(a, b, preferred_element_type=jnp.float32, precision=_HI)


def _uv_body(a_ref, wd_ref, wb_ref, u_ref, v_ref):
    a = a_ref[...]
    u_ref[...] = _dot(a, wd_ref[...])
    v_ref[...] = _dot(a, wb_ref[...])


def _tc_uv(a, wd, wb):
    n, k = a.shape
    return pl.pallas_call(
        _uv_body,
        grid=(n // BLK,),
        in_specs=[pl.BlockSpec((BLK, k), lambda i: (i, 0)),
                  pl.BlockSpec((k, 64), lambda i: (0, 0)),
                  pl.BlockSpec((k, 64), lambda i: (0, 0))],
        out_specs=[pl.BlockSpec((BLK, 64), lambda i: (i, 0)),
                   pl.BlockSpec((BLK, 64), lambda i: (i, 0))],
        out_shape=[jax.ShapeDtypeStruct((n, 64), jnp.float32)] * 2,
    )(a, wd, wb)


def _mid_body(u_ref, m_ref, b_ref, wd_ref, wb_ref, uo_ref, vo_ref):
    a = jnp.maximum(u_ref[...] + b_ref[0:1, :] + m_ref[...], 0.0)
    uo_ref[...] = _dot(a, wd_ref[...])
    vo_ref[...] = _dot(a, wb_ref[...])


def _tc_mid(u, m, b2d, wd, wb):
    return pl.pallas_call(
        _mid_body,
        grid=(NP // BLK,),
        in_specs=[pl.BlockSpec((BLK, 64), lambda i: (i, 0)),
                  pl.BlockSpec((BLK, 64), lambda i: (i, 0)),
                  pl.BlockSpec((8, 64), lambda i: (0, 0)),
                  pl.BlockSpec((64, 64), lambda i: (0, 0)),
                  pl.BlockSpec((64, 64), lambda i: (0, 0))],
        out_specs=[pl.BlockSpec((BLK, 64), lambda i: (i, 0)),
                   pl.BlockSpec((BLK, 64), lambda i: (i, 0))],
        out_shape=[jax.ShapeDtypeStruct((NP, 64), jnp.float32)] * 2,
    )(u, m, b2d, wd, wb)


def _x5g_body(u_ref, m_ref, b_ref, x5_ref, g_ref):
    i = pl.program_id(0)
    x5 = jnp.maximum(u_ref[...] + b_ref[0:1, :] + m_ref[...], 0.0)
    x5_ref[...] = x5
    pm = jnp.broadcast_to(jnp.max(x5, axis=0, keepdims=True), (8, 64))

    @pl.when(i == 0)
    def _():
        g_ref[...] = pm

    @pl.when(i > 0)
    def _():
        g_ref[...] = jnp.maximum(g_ref[...], pm)


def _tc_x5g(u, m, b2d):
    return pl.pallas_call(
        _x5g_body,
        grid=(NP // BLK,),
        in_specs=[pl.BlockSpec((BLK, 64), lambda i: (i, 0)),
                  pl.BlockSpec((BLK, 64), lambda i: (i, 0)),
                  pl.BlockSpec((8, 64), lambda i: (0, 0))],
        out_specs=[pl.BlockSpec((BLK, 64), lambda i: (i, 0)),
                   pl.BlockSpec((8, 64), lambda i: (0, 0))],
        out_shape=[jax.ShapeDtypeStruct((NP, 64), jnp.float32),
                   jax.ShapeDtypeStruct((8, 64), jnp.float32)],
    )(u, m, b2d)


def _fin_body(x5_ref, g_ref, xp_ref, a_ref, bm_ref, bl1_ref, wl2_ref,
              bl2_ref, out_ref):
    gb = _dot(g_ref[0:1, :], bm_ref[...])
    h = jnp.maximum(_dot(x5_ref[...], a_ref[...]) + gb + bl1_ref[0:1, :], 0.0)
    out_ref[...] = xp_ref[...] + _dot(h, wl2_ref[...]) + bl2_ref[0:1, :]


def _tc_final(x5, g8, xpad, amat, bmat, bl1_2d, wl2p, bl2_2d):
    return pl.pallas_call(
        _fin_body,
        grid=(NP // BLK,),
        in_specs=[pl.BlockSpec((BLK, 64), lambda i: (i, 0)),
                  pl.BlockSpec((8, 64), lambda i: (0, 0)),
                  pl.BlockSpec((BLK, 8), lambda i: (i, 0)),
                  pl.BlockSpec((64, 128), lambda i: (0, 0)),
                  pl.BlockSpec((64, 128), lambda i: (0, 0)),
                  pl.BlockSpec((8, 128), lambda i: (0, 0)),
                  pl.BlockSpec((128, 8), lambda i: (0, 0)),
                  pl.BlockSpec((8, 8), lambda i: (0, 0))],
        out_specs=pl.BlockSpec((BLK, 8), lambda i: (i, 0)),
        out_shape=jax.ShapeDtypeStruct((NP, 8), jnp.float32),
    )(x5, g8, xpad, amat, bmat, bl1_2d, wl2p, bl2_2d)


def _segmax_body(v_hbm, src_hbm, dst_hbm, st_hbm, m_hbm,
                 idx_v, rows_v, acc_v, dstv_v, stv_v, sem):
    wid = lax.axis_index("s") * 2 + lax.axis_index("c")
    lo = wid * RPW
    hi = lo + RPW
    pltpu.sync_copy(st_hbm, stv_v)

    neg = jnp.full((16,), NEG, jnp.float32)

    @pl.loop(0, RPW)
    def _(r):
        for f in range(4):
            acc_v[r, pl.ds(f * 16, 16)] = neg

    se = stv_v[pl.ds(wid, 16)]
    s = se[0]
    e = se[1]
    c0 = s // G
    c1 = lax.div(e + G - 1, G)

    def chunk(ci, carry):
        base = ci * G
        pltpu.sync_copy(src_hbm.at[pl.ds(base, G)], idx_v)
        pltpu.sync_copy(dst_hbm.at[pl.ds(base, G)], dstv_v)
        pltpu.async_copy(v_hbm.at[idx_v], rows_v, sem).wait()

        def grp(gi, carry2):
            d16 = dstv_v[pl.ds(gi * 16, 16)]
            for lane in range(16):
                d = d16[lane]

                @pl.when((d >= lo) & (d < hi))
                def _():
                    dl = d - lo
                    ei = gi * 16 + lane
                    for f in range(4):
                        sl = pl.ds(f * 16, 16)
                        acc_v[dl, sl] = jnp.maximum(acc_v[dl, sl],
                                                    rows_v[ei, sl])

            return carry2

        lax.fori_loop(0, G // 16, grp, 0)
        return carry

    lax.fori_loop(c0, c1, chunk, 0)
    pltpu.sync_copy(acc_v, m_hbm.at[pl.ds(lo, RPW)])


def _sc_segmax(v, src_s, dst_s, starts):
    mesh = plsc.VectorSubcoreMesh(core_axis_name="c", subcore_axis_name="s")
    kfn = pl.kernel(
        _segmax_body,
        out_type=jax.ShapeDtypeStruct((NP, 64), jnp.float32),
        mesh=mesh,
        scratch_types=[
            pltpu.VMEM((G,), jnp.int32),
            pltpu.VMEM((G, 64), jnp.float32),
            pltpu.VMEM((RPW, 64), jnp.float32),
            pltpu.VMEM((G,), jnp.int32),
            pltpu.VMEM((64,), jnp.int32),
            pltpu.SemaphoreType.DMA,
        ],
        compiler_params=pltpu.CompilerParams(use_tc_tiling_on_sc=False),
    )
    return kfn(v, src_s, dst_s, starts)


def kernel(x, edge_index, W1, b1, W2, b2, W3, b3, W4, b4, W5, b5,
           Wl1, bl1, Wl2, bl2):
    f32 = jnp.float32
    src = edge_index[0].astype(jnp.int32)
    dst = edge_index[1].astype(jnp.int32)

    dst_s, src_s = lax.sort((dst, src), num_keys=1)
    bounds = jnp.arange(0, NW + 1, dtype=jnp.int32) * RPW
    starts = jnp.searchsorted(dst_s, bounds).astype(jnp.int32)
    starts = jnp.concatenate([starts, jnp.zeros((31,), jnp.int32)])

    xpad = jnp.zeros((NP, 8), f32).at[:N_NODES, :3].set(x)

    wd1 = jnp.zeros((8, 64), f32).at[:3].set(W1[:3] - W1[3:])
    wb1 = jnp.zeros((8, 64), f32).at[:3].set(W1[3:])
    u, v = _tc_uv(xpad, wd1, wb1)
    m = _sc_segmax(v, src_s, dst_s, starts)
    bprev = b1

    for (W, b) in ((W2, b2), (W3, b3), (W4, b4), (W5, b5)):
        wd = W[:64] - W[64:]
        wb = W[64:]
        b2d = jnp.broadcast_to(bprev.reshape(1, 64), (8, 64))
        u, v = _tc_mid(u, m, b2d, wd, wb)
        m = _sc_segmax(v, src_s, dst_s, starts)
        bprev = b

    b2d5 = jnp.broadcast_to(bprev.reshape(1, 64), (8, 64))
    x5, g8 = _tc_x5g(u, m, b2d5)

    amat = Wl1[:64]
    bmat = Wl1[64:]
    bl1_2d = jnp.broadcast_to(bl1.reshape(1, 128), (8, 128))
    wl2p = jnp.zeros((128, 8), f32).at[:, :3].set(Wl2)
    bl2_2d = jnp.zeros((8, 8), f32).at[:, :3].set(
        jnp.broadcast_to(bl2.reshape(1, 3), (8, 3)))
    outp = _tc_final(x5, g8, xpad, amat, bmat, bl1_2d, wl2p, bl2_2d)
    return outp[:N_NODES, :3]


# speedup vs baseline: 6.4772x; 1.4230x over previous
import jax
import jax.numpy as jnp
from jax import lax
from jax.experimental import pallas as pl
from jax.experimental.pallas import tpu as pltpu
from jax.experimental.pallas import tpu_sc as plsc

N_NODES = 50000
NW = 32
RPW = 1568
NP = NW * RPW
G = 128
NEG = -3.0e38
BLK = 512

_HI = lax.Precision.HIGHEST


def _dot(a, b):
    return jnp.dot(a, b, preferred_element_type=jnp.float32, precision=_HI)


def _uv_body(a_ref, wd_ref, wb_ref, u_ref, v_ref):
    a = a_ref[...]
    u_ref[...] = _dot(a, wd_ref[...])
    v_ref[...] = _dot(a, wb_ref[...])


def _tc_uv(a, wd, wb):
    n, k = a.shape
    return pl.pallas_call(
        _uv_body,
        grid=(n // BLK,),
        in_specs=[pl.BlockSpec((BLK, k), lambda i: (i, 0)),
                  pl.BlockSpec((k, 64), lambda i: (0, 0)),
                  pl.BlockSpec((k, 64), lambda i: (0, 0))],
        out_specs=[pl.BlockSpec((BLK, 64), lambda i: (i, 0)),
                   pl.BlockSpec((BLK, 64), lambda i: (i, 0))],
        out_shape=[jax.ShapeDtypeStruct((n, 64), jnp.float32)] * 2,
    )(a, wd, wb)


def _mid_body(u_ref, m_ref, b_ref, wd_ref, wb_ref, uo_ref, vo_ref):
    a = jnp.maximum(u_ref[...] + b_ref[0:1, :] + m_ref[...], 0.0)
    uo_ref[...] = _dot(a, wd_ref[...])
    vo_ref[...] = _dot(a, wb_ref[...])


def _tc_mid(u, m, b2d, wd, wb):
    return pl.pallas_call(
        _mid_body,
        grid=(NP // BLK,),
        in_specs=[pl.BlockSpec((BLK, 64), lambda i: (i, 0)),
                  pl.BlockSpec((BLK, 64), lambda i: (i, 0)),
                  pl.BlockSpec((8, 64), lambda i: (0, 0)),
                  pl.BlockSpec((64, 64), lambda i: (0, 0)),
                  pl.BlockSpec((64, 64), lambda i: (0, 0))],
        out_specs=[pl.BlockSpec((BLK, 64), lambda i: (i, 0)),
                   pl.BlockSpec((BLK, 64), lambda i: (i, 0))],
        out_shape=[jax.ShapeDtypeStruct((NP, 64), jnp.float32)] * 2,
    )(u, m, b2d, wd, wb)


def _x5g_body(u_ref, m_ref, b_ref, x5_ref, g_ref):
    i = pl.program_id(0)
    x5 = jnp.maximum(u_ref[...] + b_ref[0:1, :] + m_ref[...], 0.0)
    x5_ref[...] = x5
    pm = jnp.broadcast_to(jnp.max(x5, axis=0, keepdims=True), (8, 64))

    @pl.when(i == 0)
    def _():
        g_ref[...] = pm

    @pl.when(i > 0)
    def _():
        g_ref[...] = jnp.maximum(g_ref[...], pm)


def _tc_x5g(u, m, b2d):
    return pl.pallas_call(
        _x5g_body,
        grid=(NP // BLK,),
        in_specs=[pl.BlockSpec((BLK, 64), lambda i: (i, 0)),
                  pl.BlockSpec((BLK, 64), lambda i: (i, 0)),
                  pl.BlockSpec((8, 64), lambda i: (0, 0))],
        out_specs=[pl.BlockSpec((BLK, 64), lambda i: (i, 0)),
                   pl.BlockSpec((8, 64), lambda i: (0, 0))],
        out_shape=[jax.ShapeDtypeStruct((NP, 64), jnp.float32),
                   jax.ShapeDtypeStruct((8, 64), jnp.float32)],
    )(u, m, b2d)


def _fin_body(x5_ref, g_ref, xp_ref, a_ref, bm_ref, bl1_ref, wl2_ref,
              bl2_ref, out_ref):
    gb = _dot(g_ref[0:1, :], bm_ref[...])
    h = jnp.maximum(_dot(x5_ref[...], a_ref[...]) + gb + bl1_ref[0:1, :], 0.0)
    out_ref[...] = xp_ref[...] + _dot(h, wl2_ref[...]) + bl2_ref[0:1, :]


def _tc_final(x5, g8, xpad, amat, bmat, bl1_2d, wl2p, bl2_2d):
    return pl.pallas_call(
        _fin_body,
        grid=(NP // BLK,),
        in_specs=[pl.BlockSpec((BLK, 64), lambda i: (i, 0)),
                  pl.BlockSpec((8, 64), lambda i: (0, 0)),
                  pl.BlockSpec((BLK, 8), lambda i: (i, 0)),
                  pl.BlockSpec((64, 128), lambda i: (0, 0)),
                  pl.BlockSpec((64, 128), lambda i: (0, 0)),
                  pl.BlockSpec((8, 128), lambda i: (0, 0)),
                  pl.BlockSpec((128, 8), lambda i: (0, 0)),
                  pl.BlockSpec((8, 8), lambda i: (0, 0))],
        out_specs=pl.BlockSpec((BLK, 8), lambda i: (i, 0)),
        out_shape=jax.ShapeDtypeStruct((NP, 8), jnp.float32),
    )(x5, g8, xpad, amat, bmat, bl1_2d, wl2p, bl2_2d)


def _segmax_body(v_hbm, src_hbm, dst_hbm, st_hbm, m_hbm,
                 idx_v, rows_v, dstv_v, acc_v, stv_v,
                 sem_g0, sem_g1, sem_i0, sem_i1, sem_d0, sem_d1):
    wid = lax.axis_index("s") * 2 + lax.axis_index("c")
    lo = wid * RPW
    hi = lo + RPW
    pltpu.sync_copy(st_hbm, stv_v)

    neg = jnp.full((16,), NEG, jnp.float32)

    @pl.loop(0, RPW)
    def _(r):
        for f in range(4):
            acc_v[r, pl.ds(f * 16, 16)] = neg

    se = stv_v[pl.ds(wid, 16)]
    s = se[0]
    e = se[1]
    c0 = s // G
    c1 = lax.div(e + G - 1, G)

    sem_g = (sem_g0, sem_g1)
    sem_i = (sem_i0, sem_i1)
    sem_d = (sem_d0, sem_d1)

    def idx_dma(ci, b):
        return (pltpu.make_async_copy(src_hbm.at[pl.ds(ci * G, G)],
                                      idx_v.at[b], sem_i[b]),
                pltpu.make_async_copy(dst_hbm.at[pl.ds(ci * G, G)],
                                      dstv_v.at[b], sem_d[b]))

    def gather(b):
        return pltpu.make_async_copy(v_hbm.at[idx_v.at[b]], rows_v.at[b],
                                     sem_g[b])

    def process(ci, b):
        def grp(gi, carry2):
            d16 = dstv_v[b, pl.ds(gi * 16, 16)]
            for lane in range(16):
                d = d16[lane]

                @pl.when((d >= lo) & (d < hi))
                def _():
                    dl = d - lo
                    ei = gi * 16 + lane
                    for f in range(4):
                        sl = pl.ds(f * 16, 16)
                        acc_v[dl, sl] = jnp.maximum(acc_v[dl, sl],
                                                    rows_v[b, ei, sl])

            return carry2

        lax.fori_loop(0, G // 16, grp, 0)

    @pl.when(c0 < c1)
    def _():
        ia, da = idx_dma(c0, 0)
        ia.start()
        da.start()
        ia.wait()
        gather(0).start()

        @pl.when(c0 + 1 < c1)
        def _():
            ib, db = idx_dma(c0 + 1, 1)
            ib.start()
            db.start()

        def pair(t, carry):
            for b in range(2):
                ci = c0 + 2 * t + b

                @pl.when(ci < c1)
                def _():
                    gather(b).wait()
                    _, dw = idx_dma(ci, b)
                    dw.wait()

                    @pl.when(ci + 1 < c1)
                    def _():
                        iw, _ = idx_dma(ci + 1, 1 - b)
                        iw.wait()
                        gather(1 - b).start()

                    @pl.when(ci + 2 < c1)
                    def _():
                        inx, _ = idx_dma(ci + 2, b)
                        inx.start()

                    process(ci, b)

                    @pl.when(ci + 2 < c1)
                    def _():
                        _, dnx = idx_dma(ci + 2, b)
                        dnx.start()

            return carry

        npairs = lax.div(c1 - c0 + 1, 2)
        lax.fori_loop(0, npairs, pair, 0)

    pltpu.sync_copy(acc_v, m_hbm.at[pl.ds(lo, RPW)])


def _sc_segmax(v, src_s, dst_s, starts):
    mesh = plsc.VectorSubcoreMesh(core_axis_name="c", subcore_axis_name="s")
    kfn = pl.kernel(
        _segmax_body,
        out_type=jax.ShapeDtypeStruct((NP, 64), jnp.float32),
        mesh=mesh,
        scratch_types=[
            pltpu.VMEM((2, G), jnp.int32),
            pltpu.VMEM((2, G, 64), jnp.float32),
            pltpu.VMEM((2, G), jnp.int32),
            pltpu.VMEM((RPW, 64), jnp.float32),
            pltpu.VMEM((64,), jnp.int32),
            pltpu.SemaphoreType.DMA,
            pltpu.SemaphoreType.DMA,
            pltpu.SemaphoreType.DMA,
            pltpu.SemaphoreType.DMA,
            pltpu.SemaphoreType.DMA,
            pltpu.SemaphoreType.DMA,
        ],
        compiler_params=pltpu.CompilerParams(use_tc_tiling_on_sc=False),
    )
    return kfn(v, src_s, dst_s, starts)


def kernel(x, edge_index, W1, b1, W2, b2, W3, b3, W4, b4, W5, b5,
           Wl1, bl1, Wl2, bl2):
    f32 = jnp.float32
    src = edge_index[0].astype(jnp.int32)
    dst = edge_index[1].astype(jnp.int32)

    dst_s, src_s = lax.sort((dst, src), num_keys=1)
    bounds = jnp.arange(0, NW + 1, dtype=jnp.int32) * RPW
    starts = jnp.searchsorted(dst_s, bounds).astype(jnp.int32)
    starts = jnp.concatenate([starts, jnp.zeros((31,), jnp.int32)])

    xpad = jnp.zeros((NP, 8), f32).at[:N_NODES, :3].set(x)

    wd1 = jnp.zeros((8, 64), f32).at[:3].set(W1[:3] - W1[3:])
    wb1 = jnp.zeros((8, 64), f32).at[:3].set(W1[3:])
    u, v = _tc_uv(xpad, wd1, wb1)
    m = _sc_segmax(v, src_s, dst_s, starts)
    bprev = b1

    for (W, b) in ((W2, b2), (W3, b3), (W4, b4), (W5, b5)):
        wd = W[:64] - W[64:]
        wb = W[64:]
        b2d = jnp.broadcast_to(bprev.reshape(1, 64), (8, 64))
        u, v = _tc_mid(u, m, b2d, wd, wb)
        m = _sc_segmax(v, src_s, dst_s, starts)
        bprev = b

    b2d5 = jnp.broadcast_to(bprev.reshape(1, 64), (8, 64))
    x5, g8 = _tc_x5g(u, m, b2d5)

    amat = Wl1[:64]
    bmat = Wl1[64:]
    bl1_2d = jnp.broadcast_to(bl1.reshape(1, 128), (8, 128))
    wl2p = jnp.zeros((128, 8), f32).at[:, :3].set(Wl2)
    bl2_2d = jnp.zeros((8, 8), f32).at[:, :3].set(
        jnp.broadcast_to(bl2.reshape(1, 3), (8, 3)))
    outp = _tc_final(x5, g8, xpad, amat, bmat, bl1_2d, wl2p, bl2_2d)
    return outp[:N_NODES, :3]
